# Initial kernel scaffold; baseline (speedup 1.0000x reference)
import jax
import jax.numpy as jnp
from jax.experimental import pallas as pl


def kernel(feature_hashes, feature_weights, weight):
    raise NotImplementedError("write your pallas kernel here")


# speedup vs baseline: 2.4425x; 2.4425x over previous
import functools

import jax
import jax.numpy as jnp
from jax import lax
from jax.experimental import pallas as pl
from jax.experimental.pallas import tpu as pltpu
from jax.experimental.pallas import tpu_sc as plsc

B = 16384
L = 50
VOCAB = 1000000
D = 64

NC = 2
NS = 16
NW = NC * NS
LANES = 16
ND = D // LANES

BAGS_PER_W = B // NW
G = 8
NCHUNK = BAGS_PER_W // G
NBUF = 2


def _bag_kernel(fh_hbm, fw_hbm, tab_hbm, out_hbm, idx_v, rows_v, fw_v, out_v, sems):
    wid = lax.axis_index("s") * NC + lax.axis_index("c")
    bag0 = wid * BAGS_PER_W

    def issue(buf, c):
        base = bag0 + c * G
        pltpu.sync_copy(fh_hbm.at[pl.ds(base, G), :], idx_v.at[buf])
        pltpu.sync_copy(fw_hbm.at[pl.ds(base, G), :], fw_v.at[buf])
        for b in range(G):
            pltpu.async_copy(tab_hbm.at[idx_v.at[buf, b]], rows_v.at[buf, b],
                             sems.at[buf])

    def drain(buf):
        for b in range(G):
            pltpu.make_async_copy(tab_hbm.at[idx_v.at[buf, b]],
                                  rows_v.at[buf, b], sems.at[buf]).wait()

    def compute(buf, c):
        lane = lax.iota(jnp.int32, LANES)
        for b in range(G):
            bag_rows = rows_v.at[buf, b]

            bag_fw = fw_v.at[buf, b]

            def body(l, accs):
                lsp = jnp.full((LANES,), l, jnp.int32)
                w = plsc.load_gather(bag_fw, [lsp])
                return tuple(
                    accs[k] + plsc.load_gather(
                        bag_rows, [lsp, lane + k * LANES]) * w
                    for k in range(ND))

            z = jnp.zeros((LANES,), jnp.float32)
            accs = lax.fori_loop(0, L, body, (z,) * ND)

            ss = accs[0] * accs[0]
            for k in range(1, ND):
                ss = ss + accs[k] * accs[k]
            lane = lax.iota(jnp.int32, LANES)
            sv = ss
            for shift in (8, 4, 2, 1):
                sv = sv + jnp.take(sv, lane ^ shift)
            i = plsc.bitcast(sv, jnp.int32)
            y = plsc.bitcast(jnp.int32(0x5F3759DF) - (i >> 1), jnp.float32)
            for _ in range(3):
                y = y * (1.5 - 0.5 * sv * y * y)
            y = jnp.minimum(y, 1e12)
            for k in range(ND):
                out_v[b, pl.ds(k * LANES, LANES)] = accs[k] * y
        pltpu.sync_copy(out_v, out_hbm.at[pl.ds(bag0 + c * G, G), :])

    issue(0, 0)

    @pl.loop(0, NCHUNK, step=NBUF)
    def _(c0):
        for p in range(NBUF):
            c = c0 + p

            @pl.when(c + 1 < NCHUNK)
            def _():
                issue((p + 1) % NBUF, c + 1)

            drain(p)
            compute(p, c)


@jax.jit
def kernel(feature_hashes, feature_weights, weight):
    mesh = plsc.VectorSubcoreMesh(core_axis_name="c", subcore_axis_name="s")
    f = pl.kernel(
        _bag_kernel,
        out_type=jax.ShapeDtypeStruct((B, D), jnp.float32),
        mesh=mesh,
        compiler_params=pltpu.CompilerParams(
            needs_layout_passes=False, use_tc_tiling_on_sc=False),
        scratch_types=[
            pltpu.VMEM((NBUF, G, L), jnp.int32),
            pltpu.VMEM((NBUF, G, L, D), jnp.float32),
            pltpu.VMEM((NBUF, G, L), jnp.float32),
            pltpu.VMEM((G, D), jnp.float32),
            pltpu.SemaphoreType.DMA((NBUF,)),
        ],
    )
    return f(feature_hashes.astype(jnp.int32), feature_weights, weight)
